# Initial kernel scaffold; baseline (speedup 1.0000x reference)
import jax
import jax.numpy as jnp
from jax.experimental import pallas as pl


def kernel(x_num, x_cat, W_num, b_num, tables):
    raise NotImplementedError("write your pallas kernel here")


# speedup vs baseline: 1.1046x; 1.1046x over previous
import functools

import jax
import jax.numpy as jnp
from jax import lax
from jax.experimental import pallas as pl
from jax.experimental.pallas import tpu as pltpu
from jax.experimental.pallas import tpu_sc as plsc

B = 4096
N_CAT = 26
CARD = 1000
H = 768
PL_ = 2
N_NUM = 1
D = H * PL_
ROWS = 1 + N_CAT
LANES = 16


def _num_body(x_ref, w_ref, b_ref, o_ref):
    o_ref[...] = x_ref[...] * w_ref[...] + b_ref[...]


def _num_embeds(x_num, W_num, b_num):
    BLK = 256
    return pl.pallas_call(
        _num_body,
        grid=(B // BLK,),
        in_specs=[
            pl.BlockSpec((BLK, N_NUM), lambda i: (i, 0)),
            pl.BlockSpec((N_NUM, D), lambda i: (0, 0)),
            pl.BlockSpec((1, D), lambda i: (0, 0)),
        ],
        out_specs=pl.BlockSpec((BLK, D), lambda i: (i, 0)),
        out_shape=jax.ShapeDtypeStruct((B, D), jnp.float32),
    )(x_num, W_num, b_num.reshape(1, D))


def _sc_gather(tables_flat, idx, num_flat):
    mesh = plsc.VectorSubcoreMesh(core_axis_name="c", subcore_axis_name="s")

    @functools.partial(
        pl.kernel,
        out_type=jax.ShapeDtypeStruct((B * ROWS, D), jnp.float32),
        mesh=mesh,
        compiler_params=pltpu.CompilerParams(use_tc_tiling_on_sc=False),
    )
    def kern(tables_hbm, idx_hbm, num_hbm, out_hbm):
        def body(idx_vm, num_vm, o_vm):
            pltpu.sync_copy(tables_hbm.at[idx_vm.at[0]],
                            o_vm.at[pl.ds(1, N_CAT)])

            @pl.loop(0, D // LANES)
            def _(i):
                o_vm[0, pl.ds(i * LANES, LANES)] = (
                    num_vm[0, pl.ds(i * LANES, LANES)])

        pltpu.emit_pipeline(
            body,
            grid=(B,),
            in_specs=[
                pl.BlockSpec((1, N_CAT), lambda b: (b, 0)),
                pl.BlockSpec((1, D), lambda b: (b, 0)),
            ],
            out_specs=[pl.BlockSpec((ROWS, D), lambda b: (b, 0))],
            core_axis_name=("c", "s"),
            dimension_semantics=(pltpu.PARALLEL,),
        )(idx_hbm, num_hbm, out_hbm)

    return kern(tables_flat, idx, num_flat)


def kernel(x_num, x_cat, W_num, b_num, tables):
    tables_flat = tables.reshape(N_CAT * CARD, D)
    idx = x_cat + (jnp.arange(N_CAT, dtype=jnp.int32) * CARD)[None, :]
    num_flat = _num_embeds(x_num, W_num, b_num)
    out = _sc_gather(tables_flat, idx, num_flat)
    return out.reshape(B, ROWS * PL_, H)
